# baseline (speedup 1.0000x reference)
import functools

import jax
import jax.numpy as jnp
from jax import lax
from jax.experimental import pallas as pl
from jax.experimental.pallas import tpu as pltpu
from jax.experimental.pallas import tpu_sc as plsc

VOCAB = 1_000_000
EMBED_DIM = 64
BATCH = 4096
SEQ = 200
NUM_SEGS = 2 * BATCH

NC = 2
NS = 16
NW = NC * NS
SEGS_PER_W = NUM_SEGS // NW


IDX_PAD = 208


def _pool_body(
    x_hbm, table_hbm, out_hbm,
    idx0_v, idx1_v, offs0_v, offs1_v, rows0_v, rows1_v, acc_v,
    sem_i0, sem_i1, sem_g0, sem_g1,
):
    wid = lax.axis_index("s") * NC + lax.axis_index("c")
    seg0 = wid * SEGS_PER_W

    idx_v = (idx0_v, idx1_v)
    offs_v = (offs0_v, offs1_v)
    rows_v = (rows0_v, rows1_v)
    sem_i = (sem_i0, sem_i1)
    sem_g = (sem_g0, sem_g1)

    def idx_start(seg, buf):
        pltpu.async_copy(
            x_hbm.at[pl.ds((seg0 + seg) * SEQ, SEQ)],
            idx_v[buf].at[pl.ds(0, SEQ)],
            sem_i[buf],
        )

    def idx_wait(seg, buf):
        pltpu.make_async_copy(
            x_hbm.at[pl.ds((seg0 + seg) * SEQ, SEQ)],
            idx_v[buf].at[pl.ds(0, SEQ)],
            sem_i[buf],
        ).wait()

    def prep_offs(buf):
        lane = lax.broadcasted_iota(jnp.int32, (16,), 0)
        for c in range(IDX_PAD // 16):
            v = idx_v[buf][pl.ds(c * 16, 16)]
            s = lax.shift_right_logical(v, 1)
            if (c + 1) * 16 > SEQ:
                s = jnp.where(lane < SEQ - c * 16, s, 0)
            offs_v[buf][pl.ds(c * 16, 16)] = s

    def gather_start(buf):
        pltpu.async_copy(table_hbm.at[offs_v[buf]], rows_v[buf], sem_g[buf])

    def gather_wait(buf):
        pltpu.make_async_copy(
            table_hbm.at[offs_v[buf]], rows_v[buf], sem_g[buf]
        ).wait()

    def reduce_into(buf, seg):
        zero = jnp.zeros((16,), jnp.float32)
        rv = rows_v[buf]
        iv = idx_v[buf]

        def add_row(row, base, carry):
            a0, a1, a2, a3 = carry
            a0 = a0 + rv[row, pl.ds(base, 16)]
            a1 = a1 + rv[row, pl.ds(base + 16, 16)]
            a2 = a2 + rv[row, pl.ds(base + 32, 16)]
            a3 = a3 + rv[row, pl.ds(base + 48, 16)]
            return (a0, a1, a2, a3)

        def red(c, carry):
            bases = (iv[pl.ds(c * 16, 16)] & 1) * EMBED_DIM
            for r in range(16):
                carry = add_row(c * 16 + r, bases[r], carry)
            return carry

        carry = lax.fori_loop(0, SEQ // 16, red, (zero, zero, zero, zero))
        tail0 = (SEQ // 16) * 16
        bases_t = (iv[pl.ds(tail0, 16)] & 1) * EMBED_DIM
        for r in range(SEQ - tail0):
            carry = add_row(tail0 + r, bases_t[r], carry)
        a0, a1, a2, a3 = carry
        acc_v[seg, pl.ds(0, 16)] = a0
        acc_v[seg, pl.ds(16, 16)] = a1
        acc_v[seg, pl.ds(32, 16)] = a2
        acc_v[seg, pl.ds(48, 16)] = a3

    idx_start(0, 0)
    idx_wait(0, 0)
    prep_offs(0)
    gather_start(0)
    idx_start(1, 1)

    def pair_loop(p, _):
        sa = 2 * p
        sb = 2 * p + 1
        gather_wait(0)
        idx_wait(sb, 1)
        prep_offs(1)
        gather_start(1)

        reduce_into(0, sa)

        @pl.when(sa + 2 < SEGS_PER_W)
        def _():
            idx_start(sa + 2, 0)

        gather_wait(1)

        @pl.when(sa + 2 < SEGS_PER_W)
        def _():
            idx_wait(sa + 2, 0)
            prep_offs(0)
            gather_start(0)

        reduce_into(1, sb)

        @pl.when(sb + 2 < SEGS_PER_W)
        def _():
            idx_start(sb + 2, 1)

        return 0

    lax.fori_loop(0, SEGS_PER_W // 2, pair_loop, 0)
    pltpu.sync_copy(acc_v, out_hbm.at[pl.ds(seg0, SEGS_PER_W)])


_pool = functools.partial(
    pl.kernel,
    out_type=jax.ShapeDtypeStruct((NUM_SEGS, EMBED_DIM), jnp.float32),
    mesh=plsc.VectorSubcoreMesh(core_axis_name="c", subcore_axis_name="s"),
    scratch_types=[
        pltpu.VMEM((IDX_PAD,), jnp.int32),
        pltpu.VMEM((IDX_PAD,), jnp.int32),
        pltpu.VMEM((IDX_PAD,), jnp.int32),
        pltpu.VMEM((IDX_PAD,), jnp.int32),
        pltpu.VMEM((IDX_PAD, 2 * EMBED_DIM), jnp.float32),
        pltpu.VMEM((IDX_PAD, 2 * EMBED_DIM), jnp.float32),
        pltpu.VMEM((SEGS_PER_W, EMBED_DIM), jnp.float32),
        pltpu.SemaphoreType.DMA,
        pltpu.SemaphoreType.DMA,
        pltpu.SemaphoreType.DMA,
        pltpu.SemaphoreType.DMA,
    ],
)(_pool_body)


def _mlp_body(s1_ref, s2_ref, w1a_ref, w1b_ref, b1_ref, w2_ref, out_ref):
    h = (
        jnp.dot(s1_ref[...], w1a_ref[...], preferred_element_type=jnp.float32)
        + jnp.dot(s2_ref[...], w1b_ref[...], preferred_element_type=jnp.float32)
        + b1_ref[...]
    )
    h = jnp.maximum(h, 0.0)
    logits = jnp.dot(h, w2_ref[...], preferred_element_type=jnp.float32)
    m = jnp.max(logits, axis=-1, keepdims=True)
    e = jnp.exp(logits - m)
    out_ref[...] = e / jnp.sum(e, axis=-1, keepdims=True)


def _mlp(pooled, w1, b1, w2):
    blk = 512
    nblk = BATCH // blk
    grid = (nblk,)
    return pl.pallas_call(
        _mlp_body,
        grid=grid,
        in_specs=[
            pl.BlockSpec((blk, EMBED_DIM), lambda i: (i, 0)),
            pl.BlockSpec((blk, EMBED_DIM), lambda i: (i + nblk, 0)),
            pl.BlockSpec((EMBED_DIM, 128), lambda i: (0, 0)),
            pl.BlockSpec((EMBED_DIM, 128), lambda i: (1, 0)),
            pl.BlockSpec((1, 128), lambda i: (0, 0)),
            pl.BlockSpec((128, 2), lambda i: (0, 0)),
        ],
        out_specs=pl.BlockSpec((blk, 2), lambda i: (i, 0)),
        out_shape=jax.ShapeDtypeStruct((BATCH, 2), jnp.float32),
    )(pooled, pooled, w1, w1, b1, w2)


def kernel(x, table, W1, b1, W2):
    x1d = x.reshape(NUM_SEGS * SEQ)
    table2 = table.reshape(VOCAB // 2, 2 * EMBED_DIM)
    pooled = _pool(x1d, table2)
    return _mlp(pooled, W1, b1.reshape(1, 128), W2)
